# Initial kernel scaffold; baseline (speedup 1.0000x reference)
import jax
import jax.numpy as jnp
from jax.experimental import pallas as pl


def kernel(dt, edge_src, edge_dst, time_w, time_b, fc_src_W, fc_src_b, fc_dst_W, fc_dst_b, attn):
    raise NotImplementedError("write your pallas kernel here")


# speedup vs baseline: 1.3186x; 1.3186x over previous
import functools

import jax
import jax.numpy as jnp
from jax.experimental import pallas as pl

NUM_DST = 10000
NUM_EDGES = 320000
DIM_TIME = 128
DIM_OUT = 128
NUM_HEAD = 8
D_HEAD = DIM_OUT // NUM_HEAD
EPS = 1e-5
MAXNORM = 1.0 - 1e-3
BLK = 1024


def _artanh(x):
    x = jnp.clip(x, -1.0 + 1e-7, 1.0 - 1e-7)
    return 0.5 * jnp.log((1.0 + x) / (1.0 - x))


def _node_body(t_ref, wT_ref, tb_ref, W_ref, b_ref, attn_ref, S_ref, l_ref, sc_ref):
    t = t_ref[...]
    wT = wT_ref[...]
    tb = tb_ref[...]
    x = jnp.cos(t * wT + tb)
    n2 = jnp.sum(x * x, axis=-1, keepdims=True)
    n = jnp.sqrt(jnp.maximum(n2, 1e-15))
    x = jnp.where(n > MAXNORM, x / n * MAXNORM, x)
    xn2 = jnp.sum(x * x, axis=-1, keepdims=True)
    xn = jnp.maximum(jnp.sqrt(jnp.maximum(xn2, 1e-15)), EPS)
    mx = jax.lax.dot_general(x, W_ref[...], (((1,), (1,)), ((), ())),
                             preferred_element_type=jnp.float32)
    mxn2 = jnp.sum(mx * mx, axis=-1, keepdims=True)
    mxn = jnp.maximum(jnp.sqrt(jnp.maximum(mxn2, 1e-15)), EPS)
    out = jnp.tanh(mxn / xn * _artanh(xn)) * mx / mxn
    on2 = jnp.sum(out * out, axis=-1, keepdims=True)
    on = jnp.sqrt(jnp.maximum(on2, 1e-15))
    out = jnp.where(on > MAXNORM, out / on * MAXNORM, out)
    b = b_ref[...]
    x2 = jnp.sum(out * out, axis=-1, keepdims=True)
    y2 = jnp.sum(b * b, axis=-1, keepdims=True)
    xy = jnp.sum(out * b, axis=-1, keepdims=True)
    num = (1.0 + 2.0 * xy + y2) * out + (1.0 - x2) * b
    den = 1.0 + 2.0 * xy + x2 * y2
    out = num / jnp.maximum(den, 1e-15)
    fn2 = jnp.sum(out * out, axis=-1, keepdims=True)
    fn = jnp.sqrt(jnp.maximum(fn2, 1e-15))
    feat = jnp.where(fn > MAXNORM, out / fn * MAXNORM, out)
    f2 = jnp.sum(feat * feat, axis=-1, keepdims=True)
    nl = jnp.maximum(jnp.sqrt(jnp.maximum(f2, 1e-15)), EPS)
    c = _artanh(nl) / nl
    l = c * feat
    g = 1.0 / c
    es = jax.lax.dot_general(l * attn_ref[...], S_ref[...],
                             (((1,), (0,)), ((), ())),
                             preferred_element_type=jnp.float32)
    l_ref[...] = l
    sc_ref[...] = jnp.concatenate(
        [f2, g, es, jnp.zeros_like(es, shape=(es.shape[0], 6))], axis=1)


@functools.partial(jax.jit, static_argnames=("rows",))
def _node_phase(t, time_w, time_b, W, b, attn, rows):
    npad = ((rows + BLK - 1) // BLK) * BLK
    t2 = jnp.zeros((npad, 1), jnp.float32).at[:rows, 0].set(t)
    wT = time_w.reshape(1, DIM_TIME)
    tb = time_b.reshape(1, DIM_TIME)
    bv = b.reshape(1, DIM_OUT)
    attnf = attn.reshape(1, DIM_OUT)
    S = (jnp.arange(DIM_OUT)[:, None] // D_HEAD ==
         jnp.arange(NUM_HEAD)[None, :]).astype(jnp.float32)
    grid = npad // BLK
    zero_map = lambda i: (0, 0)
    l, sc = pl.pallas_call(
        _node_body,
        grid=(grid,),
        in_specs=[
            pl.BlockSpec((BLK, 1), lambda i: (i, 0)),
            pl.BlockSpec((1, DIM_TIME), zero_map),
            pl.BlockSpec((1, DIM_TIME), zero_map),
            pl.BlockSpec((DIM_OUT, DIM_TIME), zero_map),
            pl.BlockSpec((1, DIM_OUT), zero_map),
            pl.BlockSpec((1, DIM_OUT), zero_map),
            pl.BlockSpec((DIM_OUT, NUM_HEAD), zero_map),
        ],
        out_specs=[
            pl.BlockSpec((BLK, DIM_OUT), lambda i: (i, 0)),
            pl.BlockSpec((BLK, 16), lambda i: (i, 0)),
        ],
        out_shape=[
            jax.ShapeDtypeStruct((npad, DIM_OUT), jnp.float32),
            jax.ShapeDtypeStruct((npad, 16), jnp.float32),
        ],
    )(t2, wT, tb, W, bv, attnf, S)
    return l[:rows], sc[:rows]


def _seg_softmax(v, dst, num_seg):
    m = jax.ops.segment_max(v, dst, num_segments=num_seg)
    m = jnp.where(jnp.isfinite(m), m, 0.0)
    ex = jnp.exp(v - m[dst])
    s = jax.ops.segment_sum(ex, dst, num_segments=num_seg)
    return ex / jnp.maximum(s[dst], 1e-15)


def kernel(dt, edge_src, edge_dst, time_w, time_b, fc_src_W, fc_src_b,
           fc_dst_W, fc_dst_b, attn):
    D = NUM_DST
    n_all = D + NUM_EDGES
    t_all = jnp.concatenate([jnp.zeros((D,), jnp.float32), dt])
    l_src, sc_src = _node_phase(t_all, time_w, time_b, fc_src_W, fc_src_b,
                                attn, rows=n_all)
    l_dst, sc_dst = _node_phase(t_all[:D], time_w, time_b, fc_dst_W,
                                fc_dst_b, attn, rows=D)

    ls = l_src[edge_src]
    ld = l_dst[edge_dst]
    scs = sc_src[edge_src]
    scd = sc_dst[edge_dst]
    r = jnp.sum(ls * ld, axis=-1)
    x2 = scs[:, 0]
    gs = scs[:, 1]
    y2 = scd[:, 0]
    gd = scd[:, 1]
    eh = scs[:, 2:10] + scd[:, 2:10]
    xy = r * gs * gd
    al = 1.0 - 2.0 * xy + y2
    be = 1.0 - x2
    nn = al * al * x2 + be * be * y2 - 2.0 * al * be * xy
    den = jnp.maximum(1.0 - 2.0 * xy + x2 * y2, 1e-15)
    q = nn / (den * den)
    n = jnp.sqrt(jnp.maximum(q, 1e-15))
    dist = 2.0 * _artanh(n)
    dist = 1.0 / (1e-15 + dist)
    dsm = _seg_softmax(dist, edge_dst, D)
    e2 = eh * dsm[:, None]
    e2 = jnp.where(e2 >= 0, e2, 0.2 * e2)
    a = _seg_softmax(e2, edge_dst, D)
    m = ls.reshape(-1, NUM_HEAD, D_HEAD) * a[:, :, None]
    ft = jax.ops.segment_sum(m, edge_dst, num_segments=D)
    rst = ft.reshape(D, DIM_OUT)
    rn = jnp.maximum(jnp.sqrt(jnp.maximum(
        jnp.sum(rst * rst, axis=-1, keepdims=True), 1e-15)), EPS)
    return jnp.tanh(rn) * rst / rn


# speedup vs baseline: 5.2340x; 3.9694x over previous
import functools

import jax
import jax.numpy as jnp
from jax import lax
from jax.experimental import pallas as pl
from jax.experimental.pallas import tpu as pltpu
from jax.experimental.pallas import tpu_sc as plsc

NUM_DST = 10000
NUM_EDGES = 320000
DIM_TIME = 128
DIM_OUT = 128
NUM_HEAD = 8
D_HEAD = DIM_OUT // NUM_HEAD
EPS = 1e-5
MAXNORM = 1.0 - 1e-3
BLK = 1024
PACK = 144
CHUNK = 128
NWORK = 32
NCHUNK = NUM_EDGES // CHUNK
ITERS = (NCHUNK + NWORK - 1) // NWORK
DPAD = 10240


def _artanh(x):
    x = jnp.clip(x, -1.0 + 1e-7, 1.0 - 1e-7)
    return 0.5 * jnp.log((1.0 + x) / (1.0 - x))


def _node_body(t_ref, wT_ref, tb_ref, W_ref, b_ref, attn_ref, S_ref, o_ref):
    t = t_ref[...]
    x = jnp.cos(t * wT_ref[...] + tb_ref[...])
    n2 = jnp.sum(x * x, axis=-1, keepdims=True)
    n = jnp.sqrt(jnp.maximum(n2, 1e-15))
    x = jnp.where(n > MAXNORM, x / n * MAXNORM, x)
    xn2 = jnp.sum(x * x, axis=-1, keepdims=True)
    xn = jnp.maximum(jnp.sqrt(jnp.maximum(xn2, 1e-15)), EPS)
    mx = lax.dot_general(x, W_ref[...], (((1,), (1,)), ((), ())),
                         preferred_element_type=jnp.float32)
    mxn2 = jnp.sum(mx * mx, axis=-1, keepdims=True)
    mxn = jnp.maximum(jnp.sqrt(jnp.maximum(mxn2, 1e-15)), EPS)
    out = jnp.tanh(mxn / xn * _artanh(xn)) * mx / mxn
    on2 = jnp.sum(out * out, axis=-1, keepdims=True)
    on = jnp.sqrt(jnp.maximum(on2, 1e-15))
    out = jnp.where(on > MAXNORM, out / on * MAXNORM, out)
    b = b_ref[...]
    x2 = jnp.sum(out * out, axis=-1, keepdims=True)
    y2 = jnp.sum(b * b, axis=-1, keepdims=True)
    xy = jnp.sum(out * b, axis=-1, keepdims=True)
    num = (1.0 + 2.0 * xy + y2) * out + (1.0 - x2) * b
    den = 1.0 + 2.0 * xy + x2 * y2
    out = num / jnp.maximum(den, 1e-15)
    fn2 = jnp.sum(out * out, axis=-1, keepdims=True)
    fn = jnp.sqrt(jnp.maximum(fn2, 1e-15))
    feat = jnp.where(fn > MAXNORM, out / fn * MAXNORM, out)
    f2 = jnp.sum(feat * feat, axis=-1, keepdims=True)
    nl = jnp.maximum(jnp.sqrt(jnp.maximum(f2, 1e-15)), EPS)
    c = _artanh(nl) / nl
    l = c * feat
    g = 1.0 / c
    es = lax.dot_general(l * attn_ref[...], S_ref[...],
                         (((1,), (0,)), ((), ())),
                         preferred_element_type=jnp.float32)
    o_ref[...] = jnp.concatenate(
        [l, f2, g, es, jnp.zeros_like(es, shape=(es.shape[0], 6))], axis=1)


@functools.partial(jax.jit, static_argnames=("rows",))
def _node_phase(t, time_w, time_b, W, b, attn, rows):
    npad = ((rows + BLK - 1) // BLK) * BLK
    t2 = jnp.zeros((npad, 1), jnp.float32).at[:rows, 0].set(t)
    wT = time_w.reshape(1, DIM_TIME)
    tb = time_b.reshape(1, DIM_TIME)
    bv = b.reshape(1, DIM_OUT)
    attnf = attn.reshape(1, DIM_OUT)
    S = (jnp.arange(DIM_OUT)[:, None] // D_HEAD ==
         jnp.arange(NUM_HEAD)[None, :]).astype(jnp.float32)
    zero_map = lambda i: (0, 0)
    packed = pl.pallas_call(
        _node_body,
        grid=(npad // BLK,),
        in_specs=[
            pl.BlockSpec((BLK, 1), lambda i: (i, 0)),
            pl.BlockSpec((1, DIM_TIME), zero_map),
            pl.BlockSpec((1, DIM_TIME), zero_map),
            pl.BlockSpec((DIM_OUT, DIM_TIME), zero_map),
            pl.BlockSpec((1, DIM_OUT), zero_map),
            pl.BlockSpec((1, DIM_OUT), zero_map),
            pl.BlockSpec((DIM_OUT, NUM_HEAD), zero_map),
        ],
        out_specs=pl.BlockSpec((BLK, PACK), lambda i: (i, 0)),
        out_shape=jax.ShapeDtypeStruct((npad, PACK), jnp.float32),
    )(t2, wT, tb, W, bv, attnf, S)
    return packed[:rows]


def _edge_a_body(sp_ref, dp_ref, es_ref, ed_ref, q_ref, eh_ref,
                 sidx, didx, srows, drows, qbuf, ehbuf, sem_s, sem_d):
    wid = lax.axis_index("s") * 2 + lax.axis_index("c")
    iota = lax.iota(jnp.int32, 16)
    zi = jnp.zeros((16,), jnp.int32)

    def chunk(i, carry):
        cid = wid + NWORK * i

        @pl.when(cid < NCHUNK)
        def _():
            base = cid * CHUNK
            pltpu.sync_copy(es_ref.at[pl.ds(base, CHUNK)], sidx)
            pltpu.sync_copy(ed_ref.at[pl.ds(base, CHUNK)], didx)
            cps = pltpu.async_copy(sp_ref.at[sidx], srows, sem_s)
            cpd = pltpu.async_copy(dp_ref.at[didx], drows, sem_d)
            cps.wait()
            cpd.wait()

            def jstep(j, c2):
                rowv = iota + j * 16

                def kstep(k, acc):
                    for u in range(8):
                        kv = zi + (k * 8 + u)
                        sv = plsc.load_gather(srows, [rowv, kv])
                        dv = plsc.load_gather(drows, [rowv, kv])
                        acc = acc + sv * dv
                    return acc

                r = lax.fori_loop(0, 16, kstep, jnp.zeros((16,), jnp.float32))
                x2 = plsc.load_gather(srows, [rowv, zi + 128])
                gs = plsc.load_gather(srows, [rowv, zi + 129])
                y2 = plsc.load_gather(drows, [rowv, zi + 128])
                gd = plsc.load_gather(drows, [rowv, zi + 129])
                xy = r * gs * gd
                al = 1.0 - 2.0 * xy + y2
                be = 1.0 - x2
                nn = al * al * x2 + be * be * y2 - 2.0 * al * be * xy
                den = jnp.maximum(1.0 - 2.0 * xy + x2 * y2, 1e-15)
                qbuf[pl.ds(j * 16, 16)] = nn / (den * den)
                for h in range(8):
                    ev = (plsc.load_gather(srows, [rowv, zi + 130 + h]) +
                          plsc.load_gather(drows, [rowv, zi + 130 + h]))
                    plsc.store_scatter(ehbuf, [rowv, zi + h], ev)
                return c2

            lax.fori_loop(0, CHUNK // 16, jstep, 0)
            pltpu.sync_copy(qbuf, q_ref.at[pl.ds(base, CHUNK)])
            pltpu.sync_copy(ehbuf, eh_ref.at[pl.ds(base, CHUNK)])

        return carry

    lax.fori_loop(0, ITERS, chunk, 0)


_edge_a = functools.partial(
    pl.kernel,
    mesh=plsc.VectorSubcoreMesh(core_axis_name="c", subcore_axis_name="s"),
    compiler_params=pltpu.CompilerParams(use_tc_tiling_on_sc=False, needs_layout_passes=False),
    out_type=[
        jax.ShapeDtypeStruct((NUM_EDGES,), jnp.float32),
        jax.ShapeDtypeStruct((NUM_EDGES, NUM_HEAD), jnp.float32),
    ],
    scratch_types=[
        pltpu.VMEM((CHUNK,), jnp.int32),
        pltpu.VMEM((CHUNK,), jnp.int32),
        pltpu.VMEM((CHUNK, PACK), jnp.float32),
        pltpu.VMEM((CHUNK, PACK), jnp.float32),
        pltpu.VMEM((CHUNK,), jnp.float32),
        pltpu.VMEM((CHUNK, NUM_HEAD), jnp.float32),
        pltpu.SemaphoreType.DMA,
        pltpu.SemaphoreType.DMA,
    ],
)(_edge_a_body)


def _edge_b_body(sp_ref, es_ref, ed_ref, w_ref, z_ref, out_ref,
                 sidx, didx, srows, wbuf, msg, acc, sem_s):
    cidx = lax.axis_index("c")
    sidx_ax = lax.axis_index("s")
    wid = sidx_ax * 2 + cidx
    iota = lax.iota(jnp.int32, 16)
    zi = jnp.zeros((16,), jnp.int32)

    @pl.when(sidx_ax == 0)
    def _():
        pltpu.sync_copy(z_ref, acc)

    plsc.subcore_barrier()

    def chunk(i, carry):
        cid = wid + NWORK * i

        @pl.when(cid < NCHUNK)
        def _():
            base = cid * CHUNK
            pltpu.sync_copy(es_ref.at[pl.ds(base, CHUNK)], sidx)
            pltpu.sync_copy(ed_ref.at[pl.ds(base, CHUNK)], didx)
            pltpu.sync_copy(w_ref.at[pl.ds(base, CHUNK)], wbuf)
            pltpu.async_copy(sp_ref.at[sidx], srows, sem_s).wait()

            def jstep(j, c2):
                rowv = iota + j * 16
                for h in range(8):
                    wv = plsc.load_gather(wbuf, [rowv, zi + h])
                    for u in range(16):
                        kv = zi + (h * 16 + u)
                        mv = plsc.load_gather(srows, [rowv, kv]) * wv
                        plsc.store_scatter(msg, [rowv, kv], mv)
                return c2

            lax.fori_loop(0, CHUNK // 16, jstep, 0)
            pltpu.sync_copy(msg, acc.at[didx], add=True)

        return carry

    lax.fori_loop(0, ITERS, chunk, 0)
    plsc.subcore_barrier()

    @pl.when(sidx_ax == 0)
    def _():
        pltpu.sync_copy(acc, out_ref.at[cidx])


_edge_b = functools.partial(
    pl.kernel,
    mesh=plsc.VectorSubcoreMesh(core_axis_name="c", subcore_axis_name="s"),
    compiler_params=pltpu.CompilerParams(use_tc_tiling_on_sc=False, needs_layout_passes=False),
    out_type=jax.ShapeDtypeStruct((2, NUM_DST, DIM_OUT), jnp.float32),
    scratch_types=[
        pltpu.VMEM((CHUNK,), jnp.int32),
        pltpu.VMEM((CHUNK,), jnp.int32),
        pltpu.VMEM((CHUNK, PACK), jnp.float32),
        pltpu.VMEM((CHUNK, NUM_HEAD), jnp.float32),
        pltpu.VMEM((CHUNK, DIM_OUT), jnp.float32),
        pltpu.VMEM_SHARED((NUM_DST, DIM_OUT), jnp.float32),
        pltpu.SemaphoreType.DMA,
    ],
)(_edge_b_body)


def _final_body(p_ref, o_ref):
    ft = p_ref[0] + p_ref[1]
    n2 = jnp.sum(ft * ft, axis=-1, keepdims=True)
    n = jnp.maximum(jnp.sqrt(jnp.maximum(n2, 1e-15)), EPS)
    o_ref[...] = jnp.tanh(n) * ft / n


def _final_phase(partials):
    p = jnp.zeros((2, DPAD, DIM_OUT), jnp.float32).at[:, :NUM_DST].set(partials)
    out = pl.pallas_call(
        _final_body,
        grid=(DPAD // BLK,),
        in_specs=[pl.BlockSpec((2, BLK, DIM_OUT), lambda i: (0, i, 0))],
        out_specs=pl.BlockSpec((BLK, DIM_OUT), lambda i: (i, 0)),
        out_shape=jax.ShapeDtypeStruct((DPAD, DIM_OUT), jnp.float32),
    )(p)
    return out[:NUM_DST]


def _seg_softmax(v, dst, num_seg):
    m = jax.ops.segment_max(v, dst, num_segments=num_seg)
    m = jnp.where(jnp.isfinite(m), m, 0.0)
    ex = jnp.exp(v - m[dst])
    s = jax.ops.segment_sum(ex, dst, num_segments=num_seg)
    return ex / jnp.maximum(s[dst], 1e-15)


def kernel(dt, edge_src, edge_dst, time_w, time_b, fc_src_W, fc_src_b,
           fc_dst_W, fc_dst_b, attn):
    D = NUM_DST
    n_all = D + NUM_EDGES
    t_all = jnp.concatenate([jnp.zeros((D,), jnp.float32), dt])
    src_pack = _node_phase(t_all, time_w, time_b, fc_src_W, fc_src_b,
                           attn, rows=n_all)
    dst_pack = _node_phase(t_all[:D], time_w, time_b, fc_dst_W, fc_dst_b,
                           attn, rows=D)

    q, eh = _edge_a(src_pack, dst_pack, edge_src, edge_dst)

    n = jnp.sqrt(jnp.maximum(q, 1e-15))
    dist = 2.0 * jnp.arctanh(jnp.clip(n, -1.0 + 1e-7, 1.0 - 1e-7))
    dist = 1.0 / (1e-15 + dist)
    dsm = _seg_softmax(dist, edge_dst, D)
    e2 = eh * dsm[:, None]
    e2 = jnp.where(e2 >= 0, e2, 0.2 * e2)
    a = _seg_softmax(e2, edge_dst, D)

    zeros = jnp.zeros((D, DIM_OUT), jnp.float32)
    partials = _edge_b(src_pack, edge_src, edge_dst, a, zeros)
    return _final_phase(partials)


# speedup vs baseline: 5.2383x; 1.0008x over previous
import functools

import jax
import jax.numpy as jnp
from jax import lax
from jax.experimental import pallas as pl
from jax.experimental.pallas import tpu as pltpu
from jax.experimental.pallas import tpu_sc as plsc

NUM_DST = 10000
NUM_EDGES = 320000
DIM_TIME = 128
DIM_OUT = 128
NUM_HEAD = 8
D_HEAD = DIM_OUT // NUM_HEAD
EPS = 1e-5
MAXNORM = 1.0 - 1e-3
BLK = 1024
PACK = 144
CHUNK = 256
CHUNK_B = 128
NWORK = 32
NCHUNK = NUM_EDGES // CHUNK
ITERS = (NCHUNK + NWORK - 1) // NWORK
NCHUNK_B = NUM_EDGES // CHUNK_B
ITERS_B = (NCHUNK_B + NWORK - 1) // NWORK
DPAD = 10240


def _artanh(x):
    x = jnp.clip(x, -1.0 + 1e-7, 1.0 - 1e-7)
    return 0.5 * jnp.log((1.0 + x) / (1.0 - x))


def _node_body(t_ref, wT_ref, tb_ref, W_ref, b_ref, attn_ref, S_ref, o_ref):
    t = t_ref[...]
    x = jnp.cos(t * wT_ref[...] + tb_ref[...])
    n2 = jnp.sum(x * x, axis=-1, keepdims=True)
    n = jnp.sqrt(jnp.maximum(n2, 1e-15))
    x = jnp.where(n > MAXNORM, x / n * MAXNORM, x)
    xn2 = jnp.sum(x * x, axis=-1, keepdims=True)
    xn = jnp.maximum(jnp.sqrt(jnp.maximum(xn2, 1e-15)), EPS)
    mx = lax.dot_general(x, W_ref[...], (((1,), (1,)), ((), ())),
                         preferred_element_type=jnp.float32)
    mxn2 = jnp.sum(mx * mx, axis=-1, keepdims=True)
    mxn = jnp.maximum(jnp.sqrt(jnp.maximum(mxn2, 1e-15)), EPS)
    out = jnp.tanh(mxn / xn * _artanh(xn)) * mx / mxn
    on2 = jnp.sum(out * out, axis=-1, keepdims=True)
    on = jnp.sqrt(jnp.maximum(on2, 1e-15))
    out = jnp.where(on > MAXNORM, out / on * MAXNORM, out)
    b = b_ref[...]
    x2 = jnp.sum(out * out, axis=-1, keepdims=True)
    y2 = jnp.sum(b * b, axis=-1, keepdims=True)
    xy = jnp.sum(out * b, axis=-1, keepdims=True)
    num = (1.0 + 2.0 * xy + y2) * out + (1.0 - x2) * b
    den = 1.0 + 2.0 * xy + x2 * y2
    out = num / jnp.maximum(den, 1e-15)
    fn2 = jnp.sum(out * out, axis=-1, keepdims=True)
    fn = jnp.sqrt(jnp.maximum(fn2, 1e-15))
    feat = jnp.where(fn > MAXNORM, out / fn * MAXNORM, out)
    f2 = jnp.sum(feat * feat, axis=-1, keepdims=True)
    nl = jnp.maximum(jnp.sqrt(jnp.maximum(f2, 1e-15)), EPS)
    c = _artanh(nl) / nl
    l = c * feat
    g = 1.0 / c
    es = lax.dot_general(l * attn_ref[...], S_ref[...],
                         (((1,), (0,)), ((), ())),
                         preferred_element_type=jnp.float32)
    o_ref[...] = jnp.concatenate(
        [l, f2, g, es, jnp.zeros_like(es, shape=(es.shape[0], 6))], axis=1)


@functools.partial(jax.jit, static_argnames=("rows",))
def _node_phase(t, time_w, time_b, W, b, attn, rows):
    npad = ((rows + BLK - 1) // BLK) * BLK
    t2 = jnp.zeros((npad, 1), jnp.float32).at[:rows, 0].set(t)
    wT = time_w.reshape(1, DIM_TIME)
    tb = time_b.reshape(1, DIM_TIME)
    bv = b.reshape(1, DIM_OUT)
    attnf = attn.reshape(1, DIM_OUT)
    S = (jnp.arange(DIM_OUT)[:, None] // D_HEAD ==
         jnp.arange(NUM_HEAD)[None, :]).astype(jnp.float32)
    zero_map = lambda i: (0, 0)
    packed = pl.pallas_call(
        _node_body,
        grid=(npad // BLK,),
        in_specs=[
            pl.BlockSpec((BLK, 1), lambda i: (i, 0)),
            pl.BlockSpec((1, DIM_TIME), zero_map),
            pl.BlockSpec((1, DIM_TIME), zero_map),
            pl.BlockSpec((DIM_OUT, DIM_TIME), zero_map),
            pl.BlockSpec((1, DIM_OUT), zero_map),
            pl.BlockSpec((1, DIM_OUT), zero_map),
            pl.BlockSpec((DIM_OUT, NUM_HEAD), zero_map),
        ],
        out_specs=pl.BlockSpec((BLK, PACK), lambda i: (i, 0)),
        out_shape=jax.ShapeDtypeStruct((npad, PACK), jnp.float32),
    )(t2, wT, tb, W, bv, attnf, S)
    return packed[:rows]


def _edge_a_body(sp_ref, dp_ref, es_ref, ed_ref, q_ref, eh_ref,
                 sidx_a, sidx_b, didx_a, didx_b, srows, drows, qbuf, ehbuf,
                 sem_s, sem_d):
    wid = lax.axis_index("s") * 2 + lax.axis_index("c")
    iota = lax.iota(jnp.int32, 16)
    zi = jnp.zeros((16,), jnp.int32)

    def chunk(i, carry):
        cid = wid + NWORK * i

        @pl.when(cid < NCHUNK)
        def _():
            base = cid * CHUNK
            pltpu.sync_copy(es_ref.at[pl.ds(base, 128)], sidx_a)
            pltpu.sync_copy(es_ref.at[pl.ds(base + 128, 128)], sidx_b)
            pltpu.sync_copy(ed_ref.at[pl.ds(base, 128)], didx_a)
            pltpu.sync_copy(ed_ref.at[pl.ds(base + 128, 128)], didx_b)
            cps0 = pltpu.async_copy(sp_ref.at[sidx_a],
                                    srows.at[pl.ds(0, 128)], sem_s)
            cps1 = pltpu.async_copy(sp_ref.at[sidx_b],
                                    srows.at[pl.ds(128, 128)], sem_s)
            cpd0 = pltpu.async_copy(dp_ref.at[didx_a],
                                    drows.at[pl.ds(0, 128)], sem_d)
            cpd1 = pltpu.async_copy(dp_ref.at[didx_b],
                                    drows.at[pl.ds(128, 128)], sem_d)
            cps0.wait()
            cps1.wait()
            cpd0.wait()
            cpd1.wait()

            def jstep(j, c2):
                rowv = iota + j * 16

                def kstep(k, acc):
                    for u in range(8):
                        kv = zi + (k * 8 + u)
                        sv = plsc.load_gather(srows, [rowv, kv])
                        dv = plsc.load_gather(drows, [rowv, kv])
                        acc = acc + sv * dv
                    return acc

                r = lax.fori_loop(0, 16, kstep, jnp.zeros((16,), jnp.float32))
                x2 = plsc.load_gather(srows, [rowv, zi + 128])
                gs = plsc.load_gather(srows, [rowv, zi + 129])
                y2 = plsc.load_gather(drows, [rowv, zi + 128])
                gd = plsc.load_gather(drows, [rowv, zi + 129])
                xy = r * gs * gd
                al = 1.0 - 2.0 * xy + y2
                be = 1.0 - x2
                nn = al * al * x2 + be * be * y2 - 2.0 * al * be * xy
                den = jnp.maximum(1.0 - 2.0 * xy + x2 * y2, 1e-15)
                qbuf[pl.ds(j * 16, 16)] = nn / (den * den)
                for h in range(8):
                    ev = (plsc.load_gather(srows, [rowv, zi + 130 + h]) +
                          plsc.load_gather(drows, [rowv, zi + 130 + h]))
                    plsc.store_scatter(ehbuf, [rowv, zi + h], ev)
                return c2

            lax.fori_loop(0, CHUNK // 16, jstep, 0)
            pltpu.sync_copy(qbuf, q_ref.at[pl.ds(base, CHUNK)])
            pltpu.sync_copy(ehbuf, eh_ref.at[pl.ds(base, CHUNK)])

        return carry

    lax.fori_loop(0, ITERS, chunk, 0)


_edge_a = functools.partial(
    pl.kernel,
    mesh=plsc.VectorSubcoreMesh(core_axis_name="c", subcore_axis_name="s"),
    compiler_params=pltpu.CompilerParams(use_tc_tiling_on_sc=False, needs_layout_passes=False),
    out_type=[
        jax.ShapeDtypeStruct((NUM_EDGES,), jnp.float32),
        jax.ShapeDtypeStruct((NUM_EDGES, NUM_HEAD), jnp.float32),
    ],
    scratch_types=[
        pltpu.VMEM((128,), jnp.int32),
        pltpu.VMEM((128,), jnp.int32),
        pltpu.VMEM((128,), jnp.int32),
        pltpu.VMEM((128,), jnp.int32),
        pltpu.VMEM((CHUNK, PACK), jnp.float32),
        pltpu.VMEM((CHUNK, PACK), jnp.float32),
        pltpu.VMEM((CHUNK,), jnp.float32),
        pltpu.VMEM((CHUNK, NUM_HEAD), jnp.float32),
        pltpu.SemaphoreType.DMA,
        pltpu.SemaphoreType.DMA,
    ],
)(_edge_a_body)


def _edge_b_body(sp_ref, es_ref, ed_ref, w_ref, z_ref, out_ref,
                 sidx_a, didx_a, srows, wbuf, msg, acc, sem_s):
    cidx = lax.axis_index("c")
    sidx_ax = lax.axis_index("s")
    wid = sidx_ax * 2 + cidx
    iota = lax.iota(jnp.int32, 16)
    zi = jnp.zeros((16,), jnp.int32)

    @pl.when(sidx_ax == 0)
    def _():
        pltpu.sync_copy(z_ref, acc)

    plsc.subcore_barrier()

    def chunk(i, carry):
        cid = wid + NWORK * i

        @pl.when(cid < NCHUNK_B)
        def _():
            base = cid * CHUNK_B
            pltpu.sync_copy(es_ref.at[pl.ds(base, CHUNK_B)], sidx_a)
            pltpu.sync_copy(ed_ref.at[pl.ds(base, CHUNK_B)], didx_a)
            pltpu.sync_copy(w_ref.at[pl.ds(base, CHUNK_B)], wbuf)
            pltpu.async_copy(sp_ref.at[sidx_a], srows, sem_s).wait()

            def jstep(j, c2):
                rowv = iota + j * 16
                for h in range(8):
                    wv = plsc.load_gather(wbuf, [rowv, zi + h])
                    for u in range(16):
                        kv = zi + (h * 16 + u)
                        mv = plsc.load_gather(srows, [rowv, kv]) * wv
                        plsc.store_scatter(msg, [rowv, kv], mv)
                return c2

            lax.fori_loop(0, CHUNK_B // 16, jstep, 0)
            pltpu.sync_copy(msg, acc.at[didx_a], add=True)

        return carry

    lax.fori_loop(0, ITERS_B, chunk, 0)
    plsc.subcore_barrier()

    @pl.when(sidx_ax == 0)
    def _():
        pltpu.sync_copy(acc, out_ref.at[cidx])


_edge_b = functools.partial(
    pl.kernel,
    mesh=plsc.VectorSubcoreMesh(core_axis_name="c", subcore_axis_name="s"),
    compiler_params=pltpu.CompilerParams(use_tc_tiling_on_sc=False, needs_layout_passes=False),
    out_type=jax.ShapeDtypeStruct((2, NUM_DST, DIM_OUT), jnp.float32),
    scratch_types=[
        pltpu.VMEM((CHUNK_B,), jnp.int32),
        pltpu.VMEM((CHUNK_B,), jnp.int32),
        pltpu.VMEM((CHUNK_B, PACK), jnp.float32),
        pltpu.VMEM((CHUNK_B, NUM_HEAD), jnp.float32),
        pltpu.VMEM((CHUNK_B, DIM_OUT), jnp.float32),
        pltpu.VMEM_SHARED((NUM_DST, DIM_OUT), jnp.float32),
        pltpu.SemaphoreType.DMA,
    ],
)(_edge_b_body)


def _final_body(p_ref, o_ref):
    ft = p_ref[0] + p_ref[1]
    n2 = jnp.sum(ft * ft, axis=-1, keepdims=True)
    n = jnp.maximum(jnp.sqrt(jnp.maximum(n2, 1e-15)), EPS)
    o_ref[...] = jnp.tanh(n) * ft / n


def _final_phase(partials):
    p = jnp.zeros((2, DPAD, DIM_OUT), jnp.float32).at[:, :NUM_DST].set(partials)
    out = pl.pallas_call(
        _final_body,
        grid=(DPAD // BLK,),
        in_specs=[pl.BlockSpec((2, BLK, DIM_OUT), lambda i: (0, i, 0))],
        out_specs=pl.BlockSpec((BLK, DIM_OUT), lambda i: (i, 0)),
        out_shape=jax.ShapeDtypeStruct((DPAD, DIM_OUT), jnp.float32),
    )(p)
    return out[:NUM_DST]


def _seg_softmax(v, dst, num_seg):
    m = jax.ops.segment_max(v, dst, num_segments=num_seg)
    m = jnp.where(jnp.isfinite(m), m, 0.0)
    ex = jnp.exp(v - m[dst])
    s = jax.ops.segment_sum(ex, dst, num_segments=num_seg)
    return ex / jnp.maximum(s[dst], 1e-15)


def kernel(dt, edge_src, edge_dst, time_w, time_b, fc_src_W, fc_src_b,
           fc_dst_W, fc_dst_b, attn):
    D = NUM_DST
    n_all = D + NUM_EDGES
    t_all = jnp.concatenate([jnp.zeros((D,), jnp.float32), dt])
    src_pack = _node_phase(t_all, time_w, time_b, fc_src_W, fc_src_b,
                           attn, rows=n_all)
    dst_pack = _node_phase(t_all[:D], time_w, time_b, fc_dst_W, fc_dst_b,
                           attn, rows=D)

    q, eh = _edge_a(src_pack, dst_pack, edge_src, edge_dst)

    n = jnp.sqrt(jnp.maximum(q, 1e-15))
    dist = 2.0 * jnp.arctanh(jnp.clip(n, -1.0 + 1e-7, 1.0 - 1e-7))
    dist = 1.0 / (1e-15 + dist)
    dsm = _seg_softmax(dist, edge_dst, D)
    e2 = eh * dsm[:, None]
    e2 = jnp.where(e2 >= 0, e2, 0.2 * e2)
    a = _seg_softmax(e2, edge_dst, D)

    zeros = jnp.zeros((D, DIM_OUT), jnp.float32)
    partials = _edge_b(src_pack, edge_src, edge_dst, a, zeros)
    return _final_phase(partials)


# speedup vs baseline: 6.8549x; 1.3086x over previous
import functools

import jax
import jax.numpy as jnp
from jax import lax
from jax.experimental import pallas as pl
from jax.experimental.pallas import tpu as pltpu
from jax.experimental.pallas import tpu_sc as plsc

NUM_DST = 10000
NUM_EDGES = 320000
DIM_TIME = 128
DIM_OUT = 128
NUM_HEAD = 8
D_HEAD = DIM_OUT // NUM_HEAD
EPS = 1e-5
MAXNORM = 1.0 - 1e-3
BLK = 1024
PACK = 144
CHUNK = 256
CHUNK_B = 128
NWORK = 32
NCHUNK = NUM_EDGES // CHUNK
ITERS = (NCHUNK + NWORK - 1) // NWORK
NCHUNK_B = NUM_EDGES // CHUNK_B
ITERS_B = (NCHUNK_B + NWORK - 1) // NWORK
DPAD = 10240


def _artanh(x):
    x = jnp.clip(x, -1.0 + 1e-7, 1.0 - 1e-7)
    return 0.5 * jnp.log((1.0 + x) / (1.0 - x))


def _node_body(t_ref, wT_ref, tb_ref, W_ref, b_ref, attn_ref, S_ref, o_ref):
    t = t_ref[...]
    x = jnp.cos(t * wT_ref[...] + tb_ref[...])
    n2 = jnp.sum(x * x, axis=-1, keepdims=True)
    n = jnp.sqrt(jnp.maximum(n2, 1e-15))
    x = jnp.where(n > MAXNORM, x / n * MAXNORM, x)
    xn2 = jnp.sum(x * x, axis=-1, keepdims=True)
    xn = jnp.maximum(jnp.sqrt(jnp.maximum(xn2, 1e-15)), EPS)
    mx = lax.dot_general(x, W_ref[...], (((1,), (1,)), ((), ())),
                         preferred_element_type=jnp.float32)
    mxn2 = jnp.sum(mx * mx, axis=-1, keepdims=True)
    mxn = jnp.maximum(jnp.sqrt(jnp.maximum(mxn2, 1e-15)), EPS)
    out = jnp.tanh(mxn / xn * _artanh(xn)) * mx / mxn
    on2 = jnp.sum(out * out, axis=-1, keepdims=True)
    on = jnp.sqrt(jnp.maximum(on2, 1e-15))
    out = jnp.where(on > MAXNORM, out / on * MAXNORM, out)
    b = b_ref[...]
    x2 = jnp.sum(out * out, axis=-1, keepdims=True)
    y2 = jnp.sum(b * b, axis=-1, keepdims=True)
    xy = jnp.sum(out * b, axis=-1, keepdims=True)
    num = (1.0 + 2.0 * xy + y2) * out + (1.0 - x2) * b
    den = 1.0 + 2.0 * xy + x2 * y2
    out = num / jnp.maximum(den, 1e-15)
    fn2 = jnp.sum(out * out, axis=-1, keepdims=True)
    fn = jnp.sqrt(jnp.maximum(fn2, 1e-15))
    feat = jnp.where(fn > MAXNORM, out / fn * MAXNORM, out)
    f2 = jnp.sum(feat * feat, axis=-1, keepdims=True)
    nl = jnp.maximum(jnp.sqrt(jnp.maximum(f2, 1e-15)), EPS)
    c = _artanh(nl) / nl
    l = c * feat
    g = 1.0 / c
    es = lax.dot_general(l * attn_ref[...], S_ref[...],
                         (((1,), (0,)), ((), ())),
                         preferred_element_type=jnp.float32)
    o_ref[...] = jnp.concatenate(
        [l, f2, g, es, jnp.zeros_like(es, shape=(es.shape[0], 6))], axis=1)


@functools.partial(jax.jit, static_argnames=("rows",))
def _node_phase(t, time_w, time_b, W, b, attn, rows):
    npad = ((rows + BLK - 1) // BLK) * BLK
    t2 = jnp.zeros((npad, 1), jnp.float32).at[:rows, 0].set(t)
    wT = time_w.reshape(1, DIM_TIME)
    tb = time_b.reshape(1, DIM_TIME)
    bv = b.reshape(1, DIM_OUT)
    attnf = attn.reshape(1, DIM_OUT)
    S = (jnp.arange(DIM_OUT)[:, None] // D_HEAD ==
         jnp.arange(NUM_HEAD)[None, :]).astype(jnp.float32)
    zero_map = lambda i: (0, 0)
    packed = pl.pallas_call(
        _node_body,
        grid=(npad // BLK,),
        in_specs=[
            pl.BlockSpec((BLK, 1), lambda i: (i, 0)),
            pl.BlockSpec((1, DIM_TIME), zero_map),
            pl.BlockSpec((1, DIM_TIME), zero_map),
            pl.BlockSpec((DIM_OUT, DIM_TIME), zero_map),
            pl.BlockSpec((1, DIM_OUT), zero_map),
            pl.BlockSpec((1, DIM_OUT), zero_map),
            pl.BlockSpec((DIM_OUT, NUM_HEAD), zero_map),
        ],
        out_specs=pl.BlockSpec((BLK, PACK), lambda i: (i, 0)),
        out_shape=jax.ShapeDtypeStruct((npad, PACK), jnp.float32),
    )(t2, wT, tb, W, bv, attnf, S)
    return packed[:rows]


def _edge_a_body(sp_ref, dp_ref, es_ref, ed_ref, q_ref, eh_ref,
                 sidx_a, sidx_b, didx_a, didx_b, srows, drows, qbuf, ehbuf,
                 sem_s, sem_d):
    wid = lax.axis_index("s") * 2 + lax.axis_index("c")
    iota = lax.iota(jnp.int32, 16)
    zi = jnp.zeros((16,), jnp.int32)

    def chunk(i, carry):
        cid = wid + NWORK * i

        @pl.when(cid < NCHUNK)
        def _():
            base = cid * CHUNK
            pltpu.sync_copy(es_ref.at[pl.ds(base, 128)], sidx_a)
            pltpu.sync_copy(es_ref.at[pl.ds(base + 128, 128)], sidx_b)
            pltpu.sync_copy(ed_ref.at[pl.ds(base, 128)], didx_a)
            pltpu.sync_copy(ed_ref.at[pl.ds(base + 128, 128)], didx_b)
            cps0 = pltpu.async_copy(sp_ref.at[sidx_a],
                                    srows.at[pl.ds(0, 128)], sem_s)
            cps1 = pltpu.async_copy(sp_ref.at[sidx_b],
                                    srows.at[pl.ds(128, 128)], sem_s)
            cpd0 = pltpu.async_copy(dp_ref.at[didx_a],
                                    drows.at[pl.ds(0, 128)], sem_d)
            cpd1 = pltpu.async_copy(dp_ref.at[didx_b],
                                    drows.at[pl.ds(128, 128)], sem_d)
            cps0.wait()
            cps1.wait()
            cpd0.wait()
            cpd1.wait()

            def jstep(j, c2):
                rowv = iota + j * 16

                def kstep(k, acc):
                    for u in range(8):
                        kv = zi + (k * 8 + u)
                        sv = plsc.load_gather(srows, [rowv, kv])
                        dv = plsc.load_gather(drows, [rowv, kv])
                        acc = acc + sv * dv
                    return acc

                r = lax.fori_loop(0, 16, kstep, jnp.zeros((16,), jnp.float32))
                x2 = plsc.load_gather(srows, [rowv, zi + 128])
                gs = plsc.load_gather(srows, [rowv, zi + 129])
                y2 = plsc.load_gather(drows, [rowv, zi + 128])
                gd = plsc.load_gather(drows, [rowv, zi + 129])
                xy = r * gs * gd
                al = 1.0 - 2.0 * xy + y2
                be = 1.0 - x2
                nn = al * al * x2 + be * be * y2 - 2.0 * al * be * xy
                den = jnp.maximum(1.0 - 2.0 * xy + x2 * y2, 1e-15)
                qbuf[pl.ds(j * 16, 16)] = nn / (den * den)
                for h in range(8):
                    ev = (plsc.load_gather(srows, [rowv, zi + 130 + h]) +
                          plsc.load_gather(drows, [rowv, zi + 130 + h]))
                    plsc.store_scatter(ehbuf, [rowv, zi + h], ev)
                return c2

            lax.fori_loop(0, CHUNK // 16, jstep, 0)
            pltpu.sync_copy(qbuf, q_ref.at[pl.ds(base, CHUNK)])
            pltpu.sync_copy(ehbuf, eh_ref.at[pl.ds(base, CHUNK)])

        return carry

    lax.fori_loop(0, ITERS, chunk, 0)


_edge_a = functools.partial(
    pl.kernel,
    mesh=plsc.VectorSubcoreMesh(core_axis_name="c", subcore_axis_name="s"),
    compiler_params=pltpu.CompilerParams(use_tc_tiling_on_sc=False, needs_layout_passes=False),
    out_type=[
        jax.ShapeDtypeStruct((NUM_EDGES,), jnp.float32),
        jax.ShapeDtypeStruct((NUM_EDGES, NUM_HEAD), jnp.float32),
    ],
    scratch_types=[
        pltpu.VMEM((128,), jnp.int32),
        pltpu.VMEM((128,), jnp.int32),
        pltpu.VMEM((128,), jnp.int32),
        pltpu.VMEM((128,), jnp.int32),
        pltpu.VMEM((CHUNK, PACK), jnp.float32),
        pltpu.VMEM((CHUNK, PACK), jnp.float32),
        pltpu.VMEM((CHUNK,), jnp.float32),
        pltpu.VMEM((CHUNK, NUM_HEAD), jnp.float32),
        pltpu.SemaphoreType.DMA,
        pltpu.SemaphoreType.DMA,
    ],
)(_edge_a_body)


def _edge_b_body(sp_ref, es_ref, ed_ref, e2_ref, m2_ref, z_ref, out_ref,
                 sidx_a, didx_a, srows, e2buf, m2rows, msg, acc, sem_s,
                 sem_m):
    cidx = lax.axis_index("c")
    sidx_ax = lax.axis_index("s")
    wid = sidx_ax * 2 + cidx
    iota = lax.iota(jnp.int32, 16)
    zi = jnp.zeros((16,), jnp.int32)
    zf = jnp.zeros((16,), jnp.float32)

    @pl.when(sidx_ax == 0)
    def _():
        pltpu.sync_copy(z_ref, acc)

    def zstep(j, c2):
        rowv = iota + j * 16
        for p in range(8):
            plsc.store_scatter(msg, [rowv, zi + 136 + p], zf)
        return c2

    lax.fori_loop(0, CHUNK_B // 16, zstep, 0)
    plsc.subcore_barrier()

    def chunk(i, carry):
        cid = wid + NWORK * i

        @pl.when(cid < NCHUNK_B)
        def _():
            base = cid * CHUNK_B
            pltpu.sync_copy(es_ref.at[pl.ds(base, CHUNK_B)], sidx_a)
            pltpu.sync_copy(ed_ref.at[pl.ds(base, CHUNK_B)], didx_a)
            pltpu.sync_copy(e2_ref.at[pl.ds(base, CHUNK_B)], e2buf)
            cpm = pltpu.async_copy(m2_ref.at[didx_a], m2rows, sem_m)
            cps = pltpu.async_copy(sp_ref.at[sidx_a], srows, sem_s)
            cpm.wait()
            cps.wait()

            def jstep(j, c2):
                rowv = iota + j * 16
                for h in range(8):
                    e2v = plsc.load_gather(e2buf, [rowv, zi + h])
                    m2v = plsc.load_gather(m2rows, [rowv, zi + h])
                    xv = jnp.exp(e2v - m2v)
                    plsc.store_scatter(msg, [rowv, zi + 128 + h], xv)
                    for u in range(16):
                        kv = zi + (h * 16 + u)
                        mv = plsc.load_gather(srows, [rowv, kv]) * xv
                        plsc.store_scatter(msg, [rowv, kv], mv)
                return c2

            lax.fori_loop(0, CHUNK_B // 16, jstep, 0)
            pltpu.sync_copy(msg, acc.at[didx_a], add=True)

        return carry

    lax.fori_loop(0, ITERS_B, chunk, 0)
    plsc.subcore_barrier()

    @pl.when(sidx_ax == 0)
    def _():
        pltpu.sync_copy(acc, out_ref.at[cidx])


_edge_b = functools.partial(
    pl.kernel,
    mesh=plsc.VectorSubcoreMesh(core_axis_name="c", subcore_axis_name="s"),
    compiler_params=pltpu.CompilerParams(use_tc_tiling_on_sc=False, needs_layout_passes=False),
    out_type=jax.ShapeDtypeStruct((2, NUM_DST, PACK), jnp.float32),
    scratch_types=[
        pltpu.VMEM((CHUNK_B,), jnp.int32),
        pltpu.VMEM((CHUNK_B,), jnp.int32),
        pltpu.VMEM((CHUNK_B, PACK), jnp.float32),
        pltpu.VMEM((CHUNK_B, NUM_HEAD), jnp.float32),
        pltpu.VMEM((CHUNK_B, 16), jnp.float32),
        pltpu.VMEM((CHUNK_B, PACK), jnp.float32),
        pltpu.VMEM_SHARED((NUM_DST, PACK), jnp.float32),
        pltpu.SemaphoreType.DMA,
        pltpu.SemaphoreType.DMA,
    ],
)(_edge_b_body)


def _final_body(p_ref, R_ref, o_ref):
    ps = p_ref[0] + p_ref[1]
    u = ps[:, :DIM_OUT]
    s2 = ps[:, DIM_OUT:DIM_OUT + NUM_HEAD]
    s2rep = lax.dot_general(s2, R_ref[...], (((1,), (0,)), ((), ())),
                            preferred_element_type=jnp.float32)
    ft = u / jnp.maximum(s2rep, 1e-15)
    n2 = jnp.sum(ft * ft, axis=-1, keepdims=True)
    n = jnp.maximum(jnp.sqrt(jnp.maximum(n2, 1e-15)), EPS)
    o_ref[...] = jnp.tanh(n) * ft / n


def _final_phase(partials):
    p = jnp.zeros((2, DPAD, PACK), jnp.float32).at[:, :NUM_DST].set(partials)
    R = (jnp.arange(NUM_HEAD)[:, None] ==
         jnp.arange(DIM_OUT)[None, :] // D_HEAD).astype(jnp.float32)
    out = pl.pallas_call(
        _final_body,
        grid=(DPAD // BLK,),
        in_specs=[pl.BlockSpec((2, BLK, PACK), lambda i: (0, i, 0)),
                  pl.BlockSpec((NUM_HEAD, DIM_OUT), lambda i: (0, 0))],
        out_specs=pl.BlockSpec((BLK, DIM_OUT), lambda i: (i, 0)),
        out_shape=jax.ShapeDtypeStruct((DPAD, DIM_OUT), jnp.float32),
    )(p, R)
    return out[:NUM_DST]


def _seg_softmax(v, dst, num_seg):
    m = jax.ops.segment_max(v, dst, num_segments=num_seg)
    m = jnp.where(jnp.isfinite(m), m, 0.0)
    ex = jnp.exp(v - m[dst])
    s = jax.ops.segment_sum(ex, dst, num_segments=num_seg)
    return ex / jnp.maximum(s[dst], 1e-15)


def kernel(dt, edge_src, edge_dst, time_w, time_b, fc_src_W, fc_src_b,
           fc_dst_W, fc_dst_b, attn):
    D = NUM_DST
    n_all = D + NUM_EDGES
    t_all = jnp.concatenate([jnp.zeros((D,), jnp.float32), dt])
    src_pack = _node_phase(t_all, time_w, time_b, fc_src_W, fc_src_b,
                           attn, rows=n_all)
    dst_pack = _node_phase(t_all[:D], time_w, time_b, fc_dst_W, fc_dst_b,
                           attn, rows=D)

    q, eh = _edge_a(src_pack, dst_pack, edge_src, edge_dst)

    n = jnp.sqrt(jnp.maximum(q, 1e-15))
    dist = 2.0 * jnp.arctanh(jnp.clip(n, -1.0 + 1e-7, 1.0 - 1e-7))
    dist = 1.0 / (1e-15 + dist)
    m1 = jax.ops.segment_max(dist, edge_dst, num_segments=D)
    m1 = jnp.where(jnp.isfinite(m1), m1, 0.0)
    ex = jnp.exp(dist - m1[edge_dst])
    s1 = jax.ops.segment_sum(ex, edge_dst, num_segments=D)
    leh = jnp.where(eh >= 0, eh, 0.2 * eh)
    e2 = (ex / jnp.maximum(s1[edge_dst], 1e-15))[:, None] * leh
    m2 = jax.ops.segment_max(e2, edge_dst, num_segments=D)
    m2 = jnp.where(jnp.isfinite(m2), m2, 0.0)
    m2p = jnp.zeros((D, 16), jnp.float32).at[:, :NUM_HEAD].set(m2)

    zeros = jnp.zeros((D, PACK), jnp.float32)
    partials = _edge_b(src_pack, edge_src, edge_dst, e2, m2p, zeros)
    return _final_phase(partials)


# speedup vs baseline: 8.0167x; 1.1695x over previous
import functools

import jax
import jax.numpy as jnp
from jax import lax
from jax.experimental import pallas as pl
from jax.experimental.pallas import tpu as pltpu
from jax.experimental.pallas import tpu_sc as plsc

NUM_DST = 10000
NUM_EDGES = 320000
DIM_TIME = 128
DIM_OUT = 128
NUM_HEAD = 8
D_HEAD = DIM_OUT // NUM_HEAD
EPS = 1e-5
MAXNORM = 1.0 - 1e-3
BLK = 1024
PACK = 144
CHUNK = 256
CHUNK_B = 128
NWORK = 32
NCHUNK = NUM_EDGES // CHUNK
ITERS = (NCHUNK + NWORK - 1) // NWORK
NCHUNK_B = NUM_EDGES // CHUNK_B
ITERS_B = (NCHUNK_B + NWORK - 1) // NWORK
DPAD = 10240


def _artanh(x):
    x = jnp.clip(x, -1.0 + 1e-7, 1.0 - 1e-7)
    return 0.5 * jnp.log((1.0 + x) / (1.0 - x))


def _node_body(t_ref, wT_ref, tb_ref, W_ref, b_ref, attn_ref, S_ref, o_ref):
    t = t_ref[...]
    x = jnp.cos(t * wT_ref[...] + tb_ref[...])
    n2 = jnp.sum(x * x, axis=-1, keepdims=True)
    n = jnp.sqrt(jnp.maximum(n2, 1e-15))
    x = jnp.where(n > MAXNORM, x / n * MAXNORM, x)
    xn2 = jnp.sum(x * x, axis=-1, keepdims=True)
    xn = jnp.maximum(jnp.sqrt(jnp.maximum(xn2, 1e-15)), EPS)
    mx = lax.dot_general(x, W_ref[...], (((1,), (1,)), ((), ())),
                         preferred_element_type=jnp.float32)
    mxn2 = jnp.sum(mx * mx, axis=-1, keepdims=True)
    mxn = jnp.maximum(jnp.sqrt(jnp.maximum(mxn2, 1e-15)), EPS)
    out = jnp.tanh(mxn / xn * _artanh(xn)) * mx / mxn
    on2 = jnp.sum(out * out, axis=-1, keepdims=True)
    on = jnp.sqrt(jnp.maximum(on2, 1e-15))
    out = jnp.where(on > MAXNORM, out / on * MAXNORM, out)
    b = b_ref[...]
    x2 = jnp.sum(out * out, axis=-1, keepdims=True)
    y2 = jnp.sum(b * b, axis=-1, keepdims=True)
    xy = jnp.sum(out * b, axis=-1, keepdims=True)
    num = (1.0 + 2.0 * xy + y2) * out + (1.0 - x2) * b
    den = 1.0 + 2.0 * xy + x2 * y2
    out = num / jnp.maximum(den, 1e-15)
    fn2 = jnp.sum(out * out, axis=-1, keepdims=True)
    fn = jnp.sqrt(jnp.maximum(fn2, 1e-15))
    feat = jnp.where(fn > MAXNORM, out / fn * MAXNORM, out)
    f2 = jnp.sum(feat * feat, axis=-1, keepdims=True)
    nl = jnp.maximum(jnp.sqrt(jnp.maximum(f2, 1e-15)), EPS)
    c = _artanh(nl) / nl
    l = c * feat
    g = 1.0 / c
    es = lax.dot_general(l * attn_ref[...], S_ref[...],
                         (((1,), (0,)), ((), ())),
                         preferred_element_type=jnp.float32)
    o_ref[...] = jnp.concatenate(
        [l, f2, g, es, jnp.zeros_like(es, shape=(es.shape[0], 6))], axis=1)


@functools.partial(jax.jit, static_argnames=("rows",))
def _node_phase(t, time_w, time_b, W, b, attn, rows):
    npad = ((rows + BLK - 1) // BLK) * BLK
    t2 = jnp.zeros((npad, 1), jnp.float32).at[:rows, 0].set(t)
    wT = time_w.reshape(1, DIM_TIME)
    tb = time_b.reshape(1, DIM_TIME)
    bv = b.reshape(1, DIM_OUT)
    attnf = attn.reshape(1, DIM_OUT)
    S = (jnp.arange(DIM_OUT)[:, None] // D_HEAD ==
         jnp.arange(NUM_HEAD)[None, :]).astype(jnp.float32)
    zero_map = lambda i: (0, 0)
    packed = pl.pallas_call(
        _node_body,
        grid=(npad // BLK,),
        in_specs=[
            pl.BlockSpec((BLK, 1), lambda i: (i, 0)),
            pl.BlockSpec((1, DIM_TIME), zero_map),
            pl.BlockSpec((1, DIM_TIME), zero_map),
            pl.BlockSpec((DIM_OUT, DIM_TIME), zero_map),
            pl.BlockSpec((1, DIM_OUT), zero_map),
            pl.BlockSpec((1, DIM_OUT), zero_map),
            pl.BlockSpec((DIM_OUT, NUM_HEAD), zero_map),
        ],
        out_specs=pl.BlockSpec((BLK, PACK), lambda i: (i, 0)),
        out_shape=jax.ShapeDtypeStruct((npad, PACK), jnp.float32),
    )(t2, wT, tb, W, bv, attnf, S)
    return packed[:rows]


def _edge_a_body(sp_ref, dp_ref, es_ref, ed_ref, q_ref, eh_ref,
                 sidx_a, sidx_b, didx_a, didx_b, srows, drows, qbuf, ehbuf,
                 sem_s, sem_d):
    wid = lax.axis_index("s") * 2 + lax.axis_index("c")
    iota = lax.iota(jnp.int32, 16)
    zi = jnp.zeros((16,), jnp.int32)

    def chunk(i, carry):
        cid = wid + NWORK * i

        @pl.when(cid < NCHUNK)
        def _():
            base = cid * CHUNK
            pltpu.sync_copy(es_ref.at[pl.ds(base, 128)], sidx_a)
            pltpu.sync_copy(es_ref.at[pl.ds(base + 128, 128)], sidx_b)
            pltpu.sync_copy(ed_ref.at[pl.ds(base, 128)], didx_a)
            pltpu.sync_copy(ed_ref.at[pl.ds(base + 128, 128)], didx_b)
            cps0 = pltpu.async_copy(sp_ref.at[sidx_a],
                                    srows.at[pl.ds(0, 128)], sem_s)
            cps1 = pltpu.async_copy(sp_ref.at[sidx_b],
                                    srows.at[pl.ds(128, 128)], sem_s)
            cpd0 = pltpu.async_copy(dp_ref.at[didx_a],
                                    drows.at[pl.ds(0, 128)], sem_d)
            cpd1 = pltpu.async_copy(dp_ref.at[didx_b],
                                    drows.at[pl.ds(128, 128)], sem_d)
            cps0.wait()
            cps1.wait()
            cpd0.wait()
            cpd1.wait()

            def jstep(j, c2):
                rowv = iota + j * 16

                def kstep(k, acc):
                    for u in range(8):
                        kv = zi + (k * 8 + u)
                        sv = plsc.load_gather(srows, [rowv, kv])
                        dv = plsc.load_gather(drows, [rowv, kv])
                        acc = acc + sv * dv
                    return acc

                r = lax.fori_loop(0, 16, kstep, jnp.zeros((16,), jnp.float32))
                x2 = plsc.load_gather(srows, [rowv, zi + 128])
                gs = plsc.load_gather(srows, [rowv, zi + 129])
                y2 = plsc.load_gather(drows, [rowv, zi + 128])
                gd = plsc.load_gather(drows, [rowv, zi + 129])
                xy = r * gs * gd
                al = 1.0 - 2.0 * xy + y2
                be = 1.0 - x2
                nn = al * al * x2 + be * be * y2 - 2.0 * al * be * xy
                den = jnp.maximum(1.0 - 2.0 * xy + x2 * y2, 1e-15)
                qbuf[pl.ds(j * 16, 16)] = nn / (den * den)
                for h in range(8):
                    ev = (plsc.load_gather(srows, [rowv, zi + 130 + h]) +
                          plsc.load_gather(drows, [rowv, zi + 130 + h]))
                    plsc.store_scatter(ehbuf, [rowv, zi + h], ev)
                return c2

            lax.fori_loop(0, CHUNK // 16, jstep, 0)
            pltpu.sync_copy(qbuf, q_ref.at[pl.ds(base, CHUNK)])
            pltpu.sync_copy(ehbuf, eh_ref.at[pl.ds(base, CHUNK)])

        return carry

    lax.fori_loop(0, ITERS, chunk, 0)


_edge_a = functools.partial(
    pl.kernel,
    mesh=plsc.VectorSubcoreMesh(core_axis_name="c", subcore_axis_name="s"),
    compiler_params=pltpu.CompilerParams(use_tc_tiling_on_sc=False, needs_layout_passes=False),
    out_type=[
        jax.ShapeDtypeStruct((NUM_EDGES,), jnp.float32),
        jax.ShapeDtypeStruct((NUM_EDGES, NUM_HEAD), jnp.float32),
    ],
    scratch_types=[
        pltpu.VMEM((128,), jnp.int32),
        pltpu.VMEM((128,), jnp.int32),
        pltpu.VMEM((128,), jnp.int32),
        pltpu.VMEM((128,), jnp.int32),
        pltpu.VMEM((CHUNK, PACK), jnp.float32),
        pltpu.VMEM((CHUNK, PACK), jnp.float32),
        pltpu.VMEM((CHUNK,), jnp.float32),
        pltpu.VMEM((CHUNK, NUM_HEAD), jnp.float32),
        pltpu.SemaphoreType.DMA,
        pltpu.SemaphoreType.DMA,
    ],
)(_edge_a_body)


def _edge_b_body(sp_ref, es_ref, ed_ref, e2_ref, m2_ref, z_ref, out_ref,
                 sidx_a, didx_a, srows, e2buf, m2rows, msg, acc, sem_s,
                 sem_m):
    cidx = lax.axis_index("c")
    sidx_ax = lax.axis_index("s")
    wid = sidx_ax * 2 + cidx
    iota = lax.iota(jnp.int32, 16)
    zi = jnp.zeros((16,), jnp.int32)
    zf = jnp.zeros((16,), jnp.float32)

    @pl.when(sidx_ax == 0)
    def _():
        pltpu.sync_copy(z_ref, acc)

    def zstep(j, c2):
        rowv = iota + j * 16
        for p in range(8):
            plsc.store_scatter(msg, [rowv, zi + 136 + p], zf)
        return c2

    lax.fori_loop(0, CHUNK_B // 16, zstep, 0)
    plsc.subcore_barrier()

    def chunk(i, carry):
        cid = wid + NWORK * i

        @pl.when(cid < NCHUNK_B)
        def _():
            base = cid * CHUNK_B
            pltpu.sync_copy(es_ref.at[pl.ds(base, CHUNK_B)], sidx_a)
            pltpu.sync_copy(ed_ref.at[pl.ds(base, CHUNK_B)], didx_a)
            pltpu.sync_copy(e2_ref.at[pl.ds(base, CHUNK_B)], e2buf)
            cpm = pltpu.async_copy(m2_ref.at[didx_a], m2rows, sem_m)
            cps = pltpu.async_copy(sp_ref.at[sidx_a], srows, sem_s)
            cpm.wait()
            cps.wait()

            def jstep(j, c2):
                rowv = iota + j * 16
                s1v = plsc.load_gather(m2rows, [rowv, zi + 8])
                rcp = 1.0 / jnp.maximum(s1v, 1e-15)
                for h in range(8):
                    e2v = plsc.load_gather(e2buf, [rowv, zi + h])
                    m2v = plsc.load_gather(m2rows, [rowv, zi + h])
                    xv = jnp.exp((e2v - m2v) * rcp)
                    plsc.store_scatter(msg, [rowv, zi + 128 + h], xv)
                    for u in range(16):
                        kv = zi + (h * 16 + u)
                        mv = plsc.load_gather(srows, [rowv, kv]) * xv
                        plsc.store_scatter(msg, [rowv, kv], mv)
                return c2

            lax.fori_loop(0, CHUNK_B // 16, jstep, 0)
            pltpu.sync_copy(msg, acc.at[didx_a], add=True)

        return carry

    lax.fori_loop(0, ITERS_B, chunk, 0)
    plsc.subcore_barrier()

    @pl.when(sidx_ax == 0)
    def _():
        pltpu.sync_copy(acc, out_ref.at[cidx])


_edge_b = functools.partial(
    pl.kernel,
    mesh=plsc.VectorSubcoreMesh(core_axis_name="c", subcore_axis_name="s"),
    compiler_params=pltpu.CompilerParams(use_tc_tiling_on_sc=False, needs_layout_passes=False),
    out_type=jax.ShapeDtypeStruct((2, NUM_DST, PACK), jnp.float32),
    scratch_types=[
        pltpu.VMEM((CHUNK_B,), jnp.int32),
        pltpu.VMEM((CHUNK_B,), jnp.int32),
        pltpu.VMEM((CHUNK_B, PACK), jnp.float32),
        pltpu.VMEM((CHUNK_B, NUM_HEAD), jnp.float32),
        pltpu.VMEM((CHUNK_B, 16), jnp.float32),
        pltpu.VMEM((CHUNK_B, PACK), jnp.float32),
        pltpu.VMEM_SHARED((NUM_DST, PACK), jnp.float32),
        pltpu.SemaphoreType.DMA,
        pltpu.SemaphoreType.DMA,
    ],
)(_edge_b_body)


def _final_body(p_ref, R_ref, o_ref):
    ps = p_ref[0] + p_ref[1]
    u = ps[:, :DIM_OUT]
    s2 = ps[:, DIM_OUT:DIM_OUT + NUM_HEAD]
    s2rep = lax.dot_general(s2, R_ref[...], (((1,), (0,)), ((), ())),
                            preferred_element_type=jnp.float32)
    ft = u / jnp.maximum(s2rep, 1e-15)
    n2 = jnp.sum(ft * ft, axis=-1, keepdims=True)
    n = jnp.maximum(jnp.sqrt(jnp.maximum(n2, 1e-15)), EPS)
    o_ref[...] = jnp.tanh(n) * ft / n


def _final_phase(partials):
    p = jnp.zeros((2, DPAD, PACK), jnp.float32).at[:, :NUM_DST].set(partials)
    R = (jnp.arange(NUM_HEAD)[:, None] ==
         jnp.arange(DIM_OUT)[None, :] // D_HEAD).astype(jnp.float32)
    out = pl.pallas_call(
        _final_body,
        grid=(DPAD // BLK,),
        in_specs=[pl.BlockSpec((2, BLK, PACK), lambda i: (0, i, 0)),
                  pl.BlockSpec((NUM_HEAD, DIM_OUT), lambda i: (0, 0))],
        out_specs=pl.BlockSpec((BLK, DIM_OUT), lambda i: (i, 0)),
        out_shape=jax.ShapeDtypeStruct((DPAD, DIM_OUT), jnp.float32),
    )(p, R)
    return out[:NUM_DST]


def _seg_softmax(v, dst, num_seg):
    m = jax.ops.segment_max(v, dst, num_segments=num_seg)
    m = jnp.where(jnp.isfinite(m), m, 0.0)
    ex = jnp.exp(v - m[dst])
    s = jax.ops.segment_sum(ex, dst, num_segments=num_seg)
    return ex / jnp.maximum(s[dst], 1e-15)


def kernel(dt, edge_src, edge_dst, time_w, time_b, fc_src_W, fc_src_b,
           fc_dst_W, fc_dst_b, attn):
    D = NUM_DST
    n_all = D + NUM_EDGES
    t_all = jnp.concatenate([jnp.zeros((D,), jnp.float32), dt])
    src_pack = _node_phase(t_all, time_w, time_b, fc_src_W, fc_src_b,
                           attn, rows=n_all)
    dst_pack = _node_phase(t_all[:D], time_w, time_b, fc_dst_W, fc_dst_b,
                           attn, rows=D)

    q, eh = _edge_a(src_pack, dst_pack, edge_src, edge_dst)

    n = jnp.sqrt(jnp.maximum(q, 1e-15))
    dist = 2.0 * jnp.arctanh(jnp.clip(n, -1.0 + 1e-7, 1.0 - 1e-7))
    dist = 1.0 / (1e-15 + dist)
    m1 = jax.ops.segment_max(dist, edge_dst, num_segments=D)
    m1 = jnp.where(jnp.isfinite(m1), m1, 0.0)
    ex = jnp.exp(dist - m1[edge_dst])
    s1 = jax.ops.segment_sum(ex, edge_dst, num_segments=D)
    leh = jnp.where(eh >= 0, eh, 0.2 * eh)
    v = ex[:, None] * leh
    m2 = jax.ops.segment_max(v, edge_dst, num_segments=D)
    m2 = jnp.where(jnp.isfinite(m2), m2, 0.0)
    m2p = (jnp.zeros((D, 16), jnp.float32).at[:, :NUM_HEAD].set(m2)
           .at[:, NUM_HEAD].set(s1))
    e2 = v

    zeros = jnp.zeros((D, PACK), jnp.float32)
    partials = _edge_b(src_pack, edge_src, edge_dst, e2, m2p, zeros)
    return _final_phase(partials)
